# baseline (speedup 1.0000x reference)
import functools

import jax
import jax.numpy as jnp
from jax import lax
from jax.experimental import pallas as pl
from jax.experimental.pallas import tpu as pltpu
from jax.experimental.pallas import tpu_sc as plsc

N_NODES = 10000
D = 128
DH = D // 2
N_EDGES = 320000

NC = 2
NS = 16

NP = 10240
ZROWS = NP // NS
K = 128
CHUNKS = 160
PHASES = 2
HCHUNK = CHUNKS // PHASES
E_PAD = NS * CHUNKS * K

_mesh = plsc.VectorSubcoreMesh(
    core_axis_name="c", subcore_axis_name="s", num_cores=NC, num_subcores=NS)


@functools.partial(
    pl.kernel,
    out_type=jax.ShapeDtypeStruct((NC, NP, DH), jnp.float32),
    mesh=_mesh,
    compiler_params=pltpu.CompilerParams(use_tc_tiling_on_sc=False),
    scratch_types=[
        pltpu.VMEM((CHUNKS, K), jnp.int32),
        pltpu.VMEM((HCHUNK, K), jnp.int32),
        pltpu.VMEM((2, K, DH), jnp.float32),
        pltpu.VMEM_SHARED((NP, DH), jnp.float32),
        pltpu.VMEM_SHARED((NP, DH), jnp.float32),
        [pltpu.SemaphoreType.DMA] * 2,
    ],
)
def _sc_aggregate(x_hbm, src_hbm, dst_hbm, zeros_hbm, out_hbm,
                  src_v, dst_v, rows_v, x_sh, acc_sh, gsem):
    c = lax.axis_index("c")
    s = lax.axis_index("s")

    rs = pl.ds(s * ZROWS, ZROWS)
    pltpu.sync_copy(x_hbm.at[rs, pl.ds(c * DH, DH)], x_sh.at[rs])
    pltpu.sync_copy(zeros_hbm, acc_sh.at[rs])
    pltpu.sync_copy(src_hbm.at[s], src_v)
    plsc.subcore_barrier()

    for p in range(PHASES):
        pltpu.sync_copy(dst_hbm.at[s, pl.ds(p * HCHUNK, HCHUNK)], dst_v)

        def round_body(g, carry, base=p * HCHUNK):
            descs = []
            for b in range(2):
                i = g * 2 + b
                descs.append(pltpu.async_copy(
                    x_sh.at[src_v.at[base + i]], rows_v.at[b], gsem[b]))
            for b in range(2):
                i = g * 2 + b
                descs[b].wait()
                pltpu.sync_copy(rows_v.at[b], acc_sh.at[dst_v.at[i]], add=True)
            return carry

        lax.fori_loop(0, HCHUNK // 2, round_body, 0)

    plsc.subcore_barrier()
    pltpu.sync_copy(acc_sh.at[rs], out_hbm.at[c, rs])


def _tc_mlp(x, agg, w1t, b1, w2t, b2, relu_out, out_rows=NP, rows=1280):
    grid = out_rows // rows

    def body(x_ref, agg_ref, w1_ref, b1_ref, w2_ref, b2_ref, o_ref):
        h = x_ref[...] + jnp.concatenate([agg_ref[0], agg_ref[1]], axis=1)
        y = jnp.dot(h, w1_ref[...], precision=lax.Precision.HIGHEST)
        y = jnp.maximum(y + b1_ref[...], 0.0)
        z = jnp.dot(y, w2_ref[...], precision=lax.Precision.HIGHEST)
        z = z + b2_ref[...]
        if relu_out:
            z = jnp.maximum(z, 0.0)
        o_ref[...] = z

    row_spec = pl.BlockSpec((rows, D), lambda i: (i, 0))
    agg_spec = pl.BlockSpec((NC, rows, DH), lambda i: (0, i, 0))
    full_spec = pl.BlockSpec((D, D), lambda i: (0, 0))
    bias_spec = pl.BlockSpec((1, D), lambda i: (0, 0))
    return pl.pallas_call(
        body,
        grid=(grid,),
        in_specs=[row_spec, agg_spec,
                  full_spec, bias_spec, full_spec, bias_spec],
        out_specs=row_spec,
        out_shape=jax.ShapeDtypeStruct((out_rows, D), jnp.float32),
    )(x, agg, w1t, b1, w2t, b2)


@jax.jit
def _run(x, edge_index, W1a, b1a, W2a, b2a, W1b, b1b, W2b, b2b):
    src = edge_index[0].astype(jnp.int32)
    dst = edge_index[1].astype(jnp.int32)
    pad = E_PAD - N_EDGES
    src_p = jnp.concatenate(
        [src, jnp.zeros((pad,), jnp.int32)]).reshape(NS, CHUNKS, K)
    dst_p = jnp.concatenate(
        [dst, jnp.full((pad,), N_NODES, jnp.int32)]).reshape(NS, CHUNKS, K)
    xp = jnp.pad(x, ((0, NP - N_NODES), (0, 0)))
    zeros_blk = jnp.zeros((ZROWS, DH), jnp.float32)

    b1a2 = b1a.reshape(1, D)
    b2a2 = b2a.reshape(1, D)
    b1b2 = b1b.reshape(1, D)
    b2b2 = b2b.reshape(1, D)

    p = _sc_aggregate(xp, src_p, dst_p, zeros_blk)
    h1 = _tc_mlp(xp, p, W1a.T, b1a2, W2a.T, b2a2, relu_out=True)
    q = _sc_aggregate(h1, src_p, dst_p, zeros_blk)
    h2 = _tc_mlp(h1, q, W1b.T, b1b2, W2b.T, b2b2, relu_out=False,
                 out_rows=N_NODES, rows=2000)
    return h2


def kernel(x, edge_index, W1a, b1a, W2a, b2a, W1b, b1b, W2b, b2b):
    return _run(x, edge_index, W1a, b1a, W2a, b2a, W1b, b1b, W2b, b2b)


# speedup vs baseline: 1.0485x; 1.0485x over previous
import functools

import jax
import jax.numpy as jnp
from jax import lax
from jax.experimental import pallas as pl
from jax.experimental.pallas import tpu as pltpu
from jax.experimental.pallas import tpu_sc as plsc

N_NODES = 10000
D = 128
DH = D // 2
N_EDGES = 320000

NC = 2
NS = 16

NP = 10240
ZROWS = NP // NS
K = 128
CHUNKS = 160
PHASES = 2
HCHUNK = CHUNKS // PHASES
E_PAD = NS * CHUNKS * K

_mesh = plsc.VectorSubcoreMesh(
    core_axis_name="c", subcore_axis_name="s", num_cores=NC, num_subcores=NS)


@functools.partial(
    pl.kernel,
    out_type=jax.ShapeDtypeStruct((NC, NP, DH), jnp.float32),
    mesh=_mesh,
    compiler_params=pltpu.CompilerParams(use_tc_tiling_on_sc=False),
    scratch_types=[
        pltpu.VMEM((CHUNKS, K), jnp.int32),
        pltpu.VMEM((HCHUNK, K), jnp.int32),
        pltpu.VMEM((K, DH), jnp.float32),
        pltpu.VMEM_SHARED((NP, DH), jnp.float32),
        pltpu.VMEM_SHARED((NP, DH), jnp.float32),
        pltpu.SemaphoreType.DMA,
    ],
)
def _sc_aggregate(x_hbm, src_hbm, dst_hbm, zeros_hbm, out_hbm,
                  src_v, dst_v, rows_v, x_sh, acc_sh, gsem):
    c = lax.axis_index("c")
    s = lax.axis_index("s")

    rs = pl.ds(s * ZROWS, ZROWS)
    pltpu.sync_copy(x_hbm.at[rs, pl.ds(c * DH, DH)], x_sh.at[rs])
    pltpu.sync_copy(zeros_hbm, acc_sh.at[rs])
    pltpu.sync_copy(src_hbm.at[s], src_v)
    plsc.subcore_barrier()

    for p in range(PHASES):
        pltpu.sync_copy(dst_hbm.at[s, pl.ds(p * HCHUNK, HCHUNK)], dst_v)

        def round_body(i, carry, base=p * HCHUNK):
            pltpu.async_copy(x_sh.at[src_v.at[base + i]], rows_v, gsem).wait()
            pltpu.sync_copy(rows_v, acc_sh.at[dst_v.at[i]], add=True)
            return carry

        lax.fori_loop(0, HCHUNK, round_body, 0)

    plsc.subcore_barrier()
    pltpu.sync_copy(acc_sh.at[rs], out_hbm.at[c, rs])


def _tc_mlp(x, agg, w1t, b1, w2t, b2, relu_out, out_rows=NP, rows=1280):
    grid = out_rows // rows

    def body(x_ref, agg_ref, w1_ref, b1_ref, w2_ref, b2_ref, o_ref):
        h = x_ref[...] + jnp.concatenate([agg_ref[0], agg_ref[1]], axis=1)
        y = jnp.dot(h, w1_ref[...], precision=lax.Precision.DEFAULT)
        y = jnp.maximum(y + b1_ref[...], 0.0)
        z = jnp.dot(y, w2_ref[...], precision=lax.Precision.DEFAULT)
        z = z + b2_ref[...]
        if relu_out:
            z = jnp.maximum(z, 0.0)
        o_ref[...] = z

    row_spec = pl.BlockSpec((rows, D), lambda i: (i, 0))
    agg_spec = pl.BlockSpec((NC, rows, DH), lambda i: (0, i, 0))
    full_spec = pl.BlockSpec((D, D), lambda i: (0, 0))
    bias_spec = pl.BlockSpec((1, D), lambda i: (0, 0))
    return pl.pallas_call(
        body,
        grid=(grid,),
        in_specs=[row_spec, agg_spec,
                  full_spec, bias_spec, full_spec, bias_spec],
        out_specs=row_spec,
        out_shape=jax.ShapeDtypeStruct((out_rows, D), jnp.float32),
    )(x, agg, w1t, b1, w2t, b2)


@jax.jit
def _run(x, edge_index, W1a, b1a, W2a, b2a, W1b, b1b, W2b, b2b):
    src = edge_index[0].astype(jnp.int32)
    dst = edge_index[1].astype(jnp.int32)
    pad = E_PAD - N_EDGES
    src_p = jnp.concatenate(
        [src, jnp.zeros((pad,), jnp.int32)]).reshape(NS, CHUNKS, K)
    dst_p = jnp.concatenate(
        [dst, jnp.full((pad,), N_NODES, jnp.int32)]).reshape(NS, CHUNKS, K)
    xp = jnp.pad(x, ((0, NP - N_NODES), (0, 0)))
    zeros_blk = jnp.zeros((ZROWS, DH), jnp.float32)

    b1a2 = b1a.reshape(1, D)
    b2a2 = b2a.reshape(1, D)
    b1b2 = b1b.reshape(1, D)
    b2b2 = b2b.reshape(1, D)

    p = _sc_aggregate(xp, src_p, dst_p, zeros_blk)
    h1 = _tc_mlp(xp, p, W1a.T, b1a2, W2a.T, b2a2, relu_out=True)
    q = _sc_aggregate(h1, src_p, dst_p, zeros_blk)
    h2 = _tc_mlp(h1, q, W1b.T, b1b2, W2b.T, b2b2, relu_out=False,
                 out_rows=N_NODES, rows=2000)
    return h2


def kernel(x, edge_index, W1a, b1a, W2a, b2a, W1b, b1b, W2b, b2b):
    return _run(x, edge_index, W1a, b1a, W2a, b2a, W1b, b1b, W2b, b2b)


# speedup vs baseline: 1.0582x; 1.0092x over previous
import functools

import jax
import jax.numpy as jnp
from jax import lax
from jax.experimental import pallas as pl
from jax.experimental.pallas import tpu as pltpu
from jax.experimental.pallas import tpu_sc as plsc

N_NODES = 10000
D = 128
DH = D // 2
N_EDGES = 320000

NC = 2
NS = 16

NP = 10240
ZROWS = NP // NS
K = 256
CHUNKS = 80
PHASES = 2
HCHUNK = CHUNKS // PHASES
E_PAD = NS * CHUNKS * K

_mesh = plsc.VectorSubcoreMesh(
    core_axis_name="c", subcore_axis_name="s", num_cores=NC, num_subcores=NS)


@functools.partial(
    pl.kernel,
    out_type=jax.ShapeDtypeStruct((NC, NP, DH), jnp.float32),
    mesh=_mesh,
    compiler_params=pltpu.CompilerParams(use_tc_tiling_on_sc=False),
    scratch_types=[
        pltpu.VMEM((CHUNKS, K), jnp.int32),
        pltpu.VMEM((HCHUNK, K), jnp.int32),
        pltpu.VMEM((K, DH), jnp.float32),
        pltpu.VMEM_SHARED((NP, DH), jnp.float32),
        pltpu.VMEM_SHARED((NP, DH), jnp.float32),
        pltpu.SemaphoreType.DMA,
    ],
)
def _sc_aggregate(x_hbm, src_hbm, dst_hbm, zeros_hbm, out_hbm,
                  src_v, dst_v, rows_v, x_sh, acc_sh, gsem):
    c = lax.axis_index("c")
    s = lax.axis_index("s")

    rs = pl.ds(s * ZROWS, ZROWS)
    pltpu.sync_copy(x_hbm.at[rs, pl.ds(c * DH, DH)], x_sh.at[rs])
    pltpu.sync_copy(zeros_hbm, acc_sh.at[rs])
    pltpu.sync_copy(src_hbm.at[s], src_v)
    plsc.subcore_barrier()

    for p in range(PHASES):
        pltpu.sync_copy(dst_hbm.at[s, pl.ds(p * HCHUNK, HCHUNK)], dst_v)

        def round_body(i, carry, base=p * HCHUNK):
            pltpu.async_copy(x_sh.at[src_v.at[base + i]], rows_v, gsem).wait()
            pltpu.sync_copy(rows_v, acc_sh.at[dst_v.at[i]], add=True)
            return carry

        lax.fori_loop(0, HCHUNK, round_body, 0)

    plsc.subcore_barrier()
    pltpu.sync_copy(acc_sh.at[rs], out_hbm.at[c, rs])


def _tc_mlp(x, agg, w1t, b1, w2t, b2, relu_out, out_rows=NP, rows=1280):
    grid = out_rows // rows

    def body(x_ref, agg_ref, w1_ref, b1_ref, w2_ref, b2_ref, o_ref):
        h = x_ref[...] + jnp.concatenate([agg_ref[0], agg_ref[1]], axis=1)
        y = jnp.dot(h, w1_ref[...], precision=lax.Precision.DEFAULT)
        y = jnp.maximum(y + b1_ref[...], 0.0)
        z = jnp.dot(y, w2_ref[...], precision=lax.Precision.DEFAULT)
        z = z + b2_ref[...]
        if relu_out:
            z = jnp.maximum(z, 0.0)
        o_ref[...] = z

    row_spec = pl.BlockSpec((rows, D), lambda i: (i, 0))
    agg_spec = pl.BlockSpec((NC, rows, DH), lambda i: (0, i, 0))
    full_spec = pl.BlockSpec((D, D), lambda i: (0, 0))
    bias_spec = pl.BlockSpec((1, D), lambda i: (0, 0))
    return pl.pallas_call(
        body,
        grid=(grid,),
        in_specs=[row_spec, agg_spec,
                  full_spec, bias_spec, full_spec, bias_spec],
        out_specs=row_spec,
        out_shape=jax.ShapeDtypeStruct((out_rows, D), jnp.float32),
    )(x, agg, w1t, b1, w2t, b2)


@jax.jit
def _run(x, edge_index, W1a, b1a, W2a, b2a, W1b, b1b, W2b, b2b):
    src = edge_index[0].astype(jnp.int32)
    dst = edge_index[1].astype(jnp.int32)
    pad = E_PAD - N_EDGES
    src_p = jnp.concatenate(
        [src, jnp.zeros((pad,), jnp.int32)]).reshape(NS, CHUNKS, K)
    dst_p = jnp.concatenate(
        [dst, jnp.full((pad,), N_NODES, jnp.int32)]).reshape(NS, CHUNKS, K)
    xp = jnp.pad(x, ((0, NP - N_NODES), (0, 0)))
    zeros_blk = jnp.zeros((ZROWS, DH), jnp.float32)

    b1a2 = b1a.reshape(1, D)
    b2a2 = b2a.reshape(1, D)
    b1b2 = b1b.reshape(1, D)
    b2b2 = b2b.reshape(1, D)

    p = _sc_aggregate(xp, src_p, dst_p, zeros_blk)
    h1 = _tc_mlp(xp, p, W1a.T, b1a2, W2a.T, b2a2, relu_out=True)
    q = _sc_aggregate(h1, src_p, dst_p, zeros_blk)
    h2 = _tc_mlp(h1, q, W1b.T, b1b2, W2b.T, b2b2, relu_out=False,
                 out_rows=N_NODES, rows=2000)
    return h2


def kernel(x, edge_index, W1a, b1a, W2a, b2a, W1b, b1b, W2b, b2b):
    return _run(x, edge_index, W1a, b1a, W2a, b2a, W1b, b1b, W2b, b2b)


# speedup vs baseline: 1.2435x; 1.1751x over previous
import functools

import jax
import jax.numpy as jnp
from jax import lax
from jax.experimental import pallas as pl
from jax.experimental.pallas import tpu as pltpu
from jax.experimental.pallas import tpu_sc as plsc

N_NODES = 10000
D = 128
DH = D // 2
N_EDGES = 320000

NC = 2
NS = 16

NP = 10240
ZROWS = NP // NS
K = 512
CHUNKS = 40
PHASES = 4
HCHUNK = CHUNKS // PHASES
E_PAD = NS * CHUNKS * K

_mesh = plsc.VectorSubcoreMesh(
    core_axis_name="c", subcore_axis_name="s", num_cores=NC, num_subcores=NS)


@functools.partial(
    pl.kernel,
    out_type=jax.ShapeDtypeStruct((NC, NP, DH), jnp.float32),
    mesh=_mesh,
    compiler_params=pltpu.CompilerParams(use_tc_tiling_on_sc=False),
    scratch_types=[
        pltpu.VMEM((HCHUNK, K), jnp.int32),
        pltpu.VMEM((HCHUNK, K), jnp.int32),
        pltpu.VMEM((K, DH), jnp.float32),
        pltpu.VMEM_SHARED((NP, DH), jnp.float32),
        pltpu.VMEM_SHARED((NP, DH), jnp.float32),
        pltpu.SemaphoreType.DMA,
    ],
)
def _sc_aggregate(x_hbm, src_hbm, dst_hbm, zeros_hbm, out_hbm,
                  src_v, dst_v, rows_v, x_sh, acc_sh, gsem):
    c = lax.axis_index("c")
    s = lax.axis_index("s")

    rs = pl.ds(s * ZROWS, ZROWS)
    pltpu.sync_copy(x_hbm.at[rs, pl.ds(c * DH, DH)], x_sh.at[rs])
    pltpu.sync_copy(zeros_hbm, acc_sh.at[rs])
    plsc.subcore_barrier()

    for p in range(PHASES):
        pltpu.sync_copy(src_hbm.at[s, pl.ds(p * HCHUNK, HCHUNK)], src_v)
        pltpu.sync_copy(dst_hbm.at[s, pl.ds(p * HCHUNK, HCHUNK)], dst_v)

        def round_body(i, carry):
            pltpu.async_copy(x_sh.at[src_v.at[i]], rows_v, gsem).wait()
            pltpu.sync_copy(rows_v, acc_sh.at[dst_v.at[i]], add=True)
            return carry

        lax.fori_loop(0, HCHUNK, round_body, 0)

    plsc.subcore_barrier()
    pltpu.sync_copy(acc_sh.at[rs], out_hbm.at[c, rs])


def _tc_mlp(x, agg, w1t, b1, w2t, b2, relu_out, out_rows=NP, rows=1280):
    grid = out_rows // rows

    def body(x_ref, agg_ref, w1_ref, b1_ref, w2_ref, b2_ref, o_ref):
        h = x_ref[...] + jnp.concatenate([agg_ref[0], agg_ref[1]], axis=1)
        y = jnp.dot(h, w1_ref[...], precision=lax.Precision.DEFAULT)
        y = jnp.maximum(y + b1_ref[...], 0.0)
        z = jnp.dot(y, w2_ref[...], precision=lax.Precision.DEFAULT)
        z = z + b2_ref[...]
        if relu_out:
            z = jnp.maximum(z, 0.0)
        o_ref[...] = z

    row_spec = pl.BlockSpec((rows, D), lambda i: (i, 0))
    agg_spec = pl.BlockSpec((NC, rows, DH), lambda i: (0, i, 0))
    full_spec = pl.BlockSpec((D, D), lambda i: (0, 0))
    bias_spec = pl.BlockSpec((1, D), lambda i: (0, 0))
    return pl.pallas_call(
        body,
        grid=(grid,),
        in_specs=[row_spec, agg_spec,
                  full_spec, bias_spec, full_spec, bias_spec],
        out_specs=row_spec,
        out_shape=jax.ShapeDtypeStruct((out_rows, D), jnp.float32),
    )(x, agg, w1t, b1, w2t, b2)


@jax.jit
def _run(x, edge_index, W1a, b1a, W2a, b2a, W1b, b1b, W2b, b2b):
    src = edge_index[0].astype(jnp.int32)
    dst = edge_index[1].astype(jnp.int32)
    pad = E_PAD - N_EDGES
    src_p = jnp.concatenate(
        [src, jnp.zeros((pad,), jnp.int32)]).reshape(NS, CHUNKS, K)
    dst_p = jnp.concatenate(
        [dst, jnp.full((pad,), N_NODES, jnp.int32)]).reshape(NS, CHUNKS, K)
    xp = jnp.pad(x, ((0, NP - N_NODES), (0, 0)))
    zeros_blk = jnp.zeros((ZROWS, DH), jnp.float32)

    b1a2 = b1a.reshape(1, D)
    b2a2 = b2a.reshape(1, D)
    b1b2 = b1b.reshape(1, D)
    b2b2 = b2b.reshape(1, D)

    p = _sc_aggregate(xp, src_p, dst_p, zeros_blk)
    h1 = _tc_mlp(xp, p, W1a.T, b1a2, W2a.T, b2a2, relu_out=True)
    q = _sc_aggregate(h1, src_p, dst_p, zeros_blk)
    h2 = _tc_mlp(h1, q, W1b.T, b1b2, W2b.T, b2b2, relu_out=False,
                 out_rows=N_NODES, rows=2000)
    return h2


def kernel(x, edge_index, W1a, b1a, W2a, b2a, W1b, b1b, W2b, b2b):
    return _run(x, edge_index, W1a, b1a, W2a, b2a, W1b, b1b, W2b, b2b)


# speedup vs baseline: 1.2855x; 1.0338x over previous
import functools

import jax
import jax.numpy as jnp
from jax import lax
from jax.experimental import pallas as pl
from jax.experimental.pallas import tpu as pltpu
from jax.experimental.pallas import tpu_sc as plsc

N_NODES = 10000
D = 128
DH = D // 2
N_EDGES = 320000

NC = 2
NS = 16

NP = 10240
ZROWS = NP // NS
K = 640
CHUNKS = 32
PHASES = 8
HCHUNK = CHUNKS // PHASES
E_PAD = NS * CHUNKS * K

_mesh = plsc.VectorSubcoreMesh(
    core_axis_name="c", subcore_axis_name="s", num_cores=NC, num_subcores=NS)


@functools.partial(
    pl.kernel,
    out_type=jax.ShapeDtypeStruct((NC, NP, DH), jnp.float32),
    mesh=_mesh,
    compiler_params=pltpu.CompilerParams(use_tc_tiling_on_sc=False),
    scratch_types=[
        pltpu.VMEM((HCHUNK, K), jnp.int32),
        pltpu.VMEM((HCHUNK, K), jnp.int32),
        pltpu.VMEM((K, DH), jnp.float32),
        pltpu.VMEM_SHARED((NP, DH), jnp.float32),
        pltpu.VMEM_SHARED((NP, DH), jnp.float32),
        pltpu.SemaphoreType.DMA,
    ],
)
def _sc_aggregate(x_hbm, src_hbm, dst_hbm, zeros_hbm, out_hbm,
                  src_v, dst_v, rows_v, x_sh, acc_sh, gsem):
    c = lax.axis_index("c")
    s = lax.axis_index("s")

    rs = pl.ds(s * ZROWS, ZROWS)
    pltpu.sync_copy(x_hbm.at[rs, pl.ds(c * DH, DH)], x_sh.at[rs])
    pltpu.sync_copy(zeros_hbm, acc_sh.at[rs])
    plsc.subcore_barrier()

    for p in range(PHASES):
        pltpu.sync_copy(src_hbm.at[s, pl.ds(p * HCHUNK, HCHUNK)], src_v)
        pltpu.sync_copy(dst_hbm.at[s, pl.ds(p * HCHUNK, HCHUNK)], dst_v)

        def round_body(i, carry):
            pltpu.async_copy(x_sh.at[src_v.at[i]], rows_v, gsem).wait()
            pltpu.sync_copy(rows_v, acc_sh.at[dst_v.at[i]], add=True)
            return carry

        lax.fori_loop(0, HCHUNK, round_body, 0)

    plsc.subcore_barrier()
    pltpu.sync_copy(acc_sh.at[rs], out_hbm.at[c, rs])


def _tc_mlp(x, agg, w1t, b1, w2t, b2, relu_out, out_rows=NP, rows=1280):
    grid = out_rows // rows

    def body(x_ref, agg_ref, w1_ref, b1_ref, w2_ref, b2_ref, o_ref):
        h = x_ref[...] + jnp.concatenate([agg_ref[0], agg_ref[1]], axis=1)
        y = jnp.dot(h, w1_ref[...], precision=lax.Precision.DEFAULT)
        y = jnp.maximum(y + b1_ref[...], 0.0)
        z = jnp.dot(y, w2_ref[...], precision=lax.Precision.DEFAULT)
        z = z + b2_ref[...]
        if relu_out:
            z = jnp.maximum(z, 0.0)
        o_ref[...] = z

    row_spec = pl.BlockSpec((rows, D), lambda i: (i, 0))
    agg_spec = pl.BlockSpec((NC, rows, DH), lambda i: (0, i, 0))
    full_spec = pl.BlockSpec((D, D), lambda i: (0, 0))
    bias_spec = pl.BlockSpec((1, D), lambda i: (0, 0))
    return pl.pallas_call(
        body,
        grid=(grid,),
        in_specs=[row_spec, agg_spec,
                  full_spec, bias_spec, full_spec, bias_spec],
        out_specs=row_spec,
        out_shape=jax.ShapeDtypeStruct((out_rows, D), jnp.float32),
    )(x, agg, w1t, b1, w2t, b2)


@jax.jit
def _run(x, edge_index, W1a, b1a, W2a, b2a, W1b, b1b, W2b, b2b):
    src = edge_index[0].astype(jnp.int32)
    dst = edge_index[1].astype(jnp.int32)
    pad = E_PAD - N_EDGES
    src_p = jnp.concatenate(
        [src, jnp.zeros((pad,), jnp.int32)]).reshape(NS, CHUNKS, K)
    dst_p = jnp.concatenate(
        [dst, jnp.full((pad,), N_NODES, jnp.int32)]).reshape(NS, CHUNKS, K)
    xp = jnp.pad(x, ((0, NP - N_NODES), (0, 0)))
    zeros_blk = jnp.zeros((ZROWS, DH), jnp.float32)

    b1a2 = b1a.reshape(1, D)
    b2a2 = b2a.reshape(1, D)
    b1b2 = b1b.reshape(1, D)
    b2b2 = b2b.reshape(1, D)

    p = _sc_aggregate(xp, src_p, dst_p, zeros_blk)
    h1 = _tc_mlp(xp, p, W1a.T, b1a2, W2a.T, b2a2, relu_out=True)
    q = _sc_aggregate(h1, src_p, dst_p, zeros_blk)
    h2 = _tc_mlp(h1, q, W1b.T, b1b2, W2b.T, b2b2, relu_out=False,
                 out_rows=N_NODES, rows=2000)
    return h2


def kernel(x, edge_index, W1a, b1a, W2a, b2a, W1b, b1b, W2b, b2b):
    return _run(x, edge_index, W1a, b1a, W2a, b2a, W1b, b1b, W2b, b2b)


# speedup vs baseline: 1.2901x; 1.0036x over previous
import functools

import jax
import jax.numpy as jnp
from jax import lax
from jax.experimental import pallas as pl
from jax.experimental.pallas import tpu as pltpu
from jax.experimental.pallas import tpu_sc as plsc

N_NODES = 10000
D = 128
DH = D // 2
N_EDGES = 320000

NC = 2
NS = 16

NP = 10240
ZROWS = NP // NS
K = 640
CHUNKS = 32
PHASES = 8
HCHUNK = CHUNKS // PHASES
E_PAD = NS * CHUNKS * K

_mesh = plsc.VectorSubcoreMesh(
    core_axis_name="c", subcore_axis_name="s", num_cores=NC, num_subcores=NS)


@functools.partial(
    pl.kernel,
    out_type=jax.ShapeDtypeStruct((NC, NP, DH), jnp.float32),
    mesh=_mesh,
    compiler_params=pltpu.CompilerParams(use_tc_tiling_on_sc=False),
    scratch_types=[
        pltpu.VMEM((2, HCHUNK, K), jnp.int32),
        pltpu.VMEM((HCHUNK, K), jnp.int32),
        pltpu.VMEM((K, DH), jnp.float32),
        pltpu.VMEM_SHARED((NP, DH), jnp.float32),
        pltpu.VMEM_SHARED((NP, DH), jnp.float32),
        pltpu.SemaphoreType.DMA,
        pltpu.SemaphoreType.DMA,
        pltpu.SemaphoreType.DMA,
    ],
)
def _sc_aggregate(x_hbm, src_hbm, dst_hbm, zeros_hbm, out_hbm,
                  src_v, dst_v, rows_v, x_sh, acc_sh, gsem, ssem, dsem):
    c = lax.axis_index("c")
    s = lax.axis_index("s")

    rs = pl.ds(s * ZROWS, ZROWS)
    pltpu.sync_copy(x_hbm.at[rs, pl.ds(c * DH, DH)], x_sh.at[rs])
    pltpu.sync_copy(zeros_hbm, acc_sh.at[rs])
    plsc.subcore_barrier()

    sdesc = None
    for p in range(PHASES):
        pb = p % 2
        if p == 0:
            pltpu.sync_copy(src_hbm.at[s, pl.ds(0, HCHUNK)], src_v.at[0])
        else:
            sdesc.wait()
        if p + 1 < PHASES:
            sdesc = pltpu.async_copy(
                src_hbm.at[s, pl.ds((p + 1) * HCHUNK, HCHUNK)],
                src_v.at[(p + 1) % 2], ssem)
        gd = pltpu.async_copy(x_sh.at[src_v.at[pb, 0]], rows_v, gsem)
        dd = pltpu.async_copy(
            dst_hbm.at[s, pl.ds(p * HCHUNK, HCHUNK)], dst_v, dsem)
        for i in range(HCHUNK):
            gd.wait()
            if i == 0:
                dd.wait()
            pltpu.sync_copy(rows_v, acc_sh.at[dst_v.at[i]], add=True)
            if i + 1 < HCHUNK:
                gd = pltpu.async_copy(
                    x_sh.at[src_v.at[pb, i + 1]], rows_v, gsem)

    plsc.subcore_barrier()
    pltpu.sync_copy(acc_sh.at[rs], out_hbm.at[c, rs])


def _tc_mlp(x, agg, w1t, b1, w2t, b2, relu_out, out_rows=NP, rows=1280):
    grid = out_rows // rows

    def body(x_ref, agg_ref, w1_ref, b1_ref, w2_ref, b2_ref, o_ref):
        h = x_ref[...] + jnp.concatenate([agg_ref[0], agg_ref[1]], axis=1)
        y = jnp.dot(h, w1_ref[...], precision=lax.Precision.DEFAULT)
        y = jnp.maximum(y + b1_ref[...], 0.0)
        z = jnp.dot(y, w2_ref[...], precision=lax.Precision.DEFAULT)
        z = z + b2_ref[...]
        if relu_out:
            z = jnp.maximum(z, 0.0)
        o_ref[...] = z

    row_spec = pl.BlockSpec((rows, D), lambda i: (i, 0))
    agg_spec = pl.BlockSpec((NC, rows, DH), lambda i: (0, i, 0))
    full_spec = pl.BlockSpec((D, D), lambda i: (0, 0))
    bias_spec = pl.BlockSpec((1, D), lambda i: (0, 0))
    return pl.pallas_call(
        body,
        grid=(grid,),
        in_specs=[row_spec, agg_spec,
                  full_spec, bias_spec, full_spec, bias_spec],
        out_specs=row_spec,
        out_shape=jax.ShapeDtypeStruct((out_rows, D), jnp.float32),
    )(x, agg, w1t, b1, w2t, b2)


@jax.jit
def _run(x, edge_index, W1a, b1a, W2a, b2a, W1b, b1b, W2b, b2b):
    src = edge_index[0].astype(jnp.int32)
    dst = edge_index[1].astype(jnp.int32)
    pad = E_PAD - N_EDGES
    src_p = jnp.concatenate(
        [src, jnp.zeros((pad,), jnp.int32)]).reshape(NS, CHUNKS, K)
    dst_p = jnp.concatenate(
        [dst, jnp.full((pad,), N_NODES, jnp.int32)]).reshape(NS, CHUNKS, K)
    xp = jnp.pad(x, ((0, NP - N_NODES), (0, 0)))
    zeros_blk = jnp.zeros((ZROWS, DH), jnp.float32)

    b1a2 = b1a.reshape(1, D)
    b2a2 = b2a.reshape(1, D)
    b1b2 = b1b.reshape(1, D)
    b2b2 = b2b.reshape(1, D)

    p = _sc_aggregate(xp, src_p, dst_p, zeros_blk)
    h1 = _tc_mlp(xp, p, W1a.T, b1a2, W2a.T, b2a2, relu_out=True)
    q = _sc_aggregate(h1, src_p, dst_p, zeros_blk)
    h2 = _tc_mlp(h1, q, W1b.T, b1b2, W2b.T, b2b2, relu_out=False,
                 out_rows=N_NODES, rows=2000)
    return h2


def kernel(x, edge_index, W1a, b1a, W2a, b2a, W1b, b1b, W2b, b2b):
    return _run(x, edge_index, W1a, b1a, W2a, b2a, W1b, b1b, W2b, b2b)
